# baseline (speedup 1.0000x reference)
import functools

import jax
import jax.numpy as jnp
from jax import lax
from jax.experimental import pallas as pl
from jax.experimental.pallas import tpu as pltpu
from jax.experimental.pallas import tpu_sc as plsc

NUM_EMB = 100000
SEM_DIM = 4
EMB_DIM = 64
PAD = NUM_EMB * SEM_DIM
MAXV = PAD - 1
B = 1024
L = 800
LF = 4

NC = 2
NS = 16
W = NC * NS
MAIN = B * L
PW = MAIN // W
C = 1024
G = 128
KC = C // G
NCH = PW // C
FUTW = (B * LF) // W


def _ids_from(s, t, m=None):
    t = jnp.minimum(jnp.maximum(t, jnp.int32(0)), jnp.int32(SEM_DIM - 1))
    ids = t * jnp.int32(NUM_EMB) + s
    inv = ((ids > jnp.int32(MAXV)) | (ids < jnp.int32(0))) & (s != jnp.int32(-1))
    ids = jnp.where(inv, jnp.int32(PAD), ids)
    if m is not None:
        ids = jnp.where(m != jnp.int32(0), ids, jnp.int32(PAD))
    return ids


@functools.partial(
    pl.kernel,
    mesh=plsc.VectorSubcoreMesh(core_axis_name="c", subcore_axis_name="s"),
    compiler_params=pltpu.CompilerParams(use_tc_tiling_on_sc=False),
    out_type=(
        jax.ShapeDtypeStruct((MAIN, EMB_DIM), jnp.float32),
        jax.ShapeDtypeStruct((B * LF, EMB_DIM), jnp.float32),
    ),
    scratch_types=[
        pltpu.VMEM((C,), jnp.int32),
        pltpu.VMEM((C,), jnp.int32),
        pltpu.VMEM((C,), jnp.int32),
        pltpu.VMEM((KC, G), jnp.int32),
        pltpu.VMEM((C, EMB_DIM), jnp.float32),
        pltpu.SemaphoreType.DMA,
    ],
)
def _sc_embed(emb, sem, tt, msk, semf, ttf, out, outf, sv, tv, mv, idx2, rows, dsem):
    wid = lax.axis_index("s") * NC + lax.axis_index("c")
    base = wid * PW

    def chunk(ci, carry):
        off = base + ci * C
        pltpu.sync_copy(sem.at[pl.ds(off, C)], sv)
        pltpu.sync_copy(tt.at[pl.ds(off, C)], tv)
        pltpu.sync_copy(msk.at[pl.ds(off, C)], mv)
        for j in range(KC):
            for k in range(G // 16):
                o = j * G + k * 16
                ids = _ids_from(sv[pl.ds(o, 16)], tv[pl.ds(o, 16)], mv[pl.ds(o, 16)])
                idx2[j, pl.ds(k * 16, 16)] = ids
        cps = [
            pltpu.async_copy(emb.at[idx2.at[j]], rows.at[pl.ds(j * G, G)], dsem)
            for j in range(KC)
        ]
        for cp in cps:
            cp.wait()
        pltpu.sync_copy(rows, out.at[pl.ds(off, C)])
        return carry

    lax.fori_loop(0, NCH, chunk, 0)

    fbase = wid * FUTW
    pltpu.sync_copy(semf.at[pl.ds(fbase, FUTW)], sv.at[pl.ds(0, FUTW)])
    pltpu.sync_copy(ttf.at[pl.ds(fbase, FUTW)], tv.at[pl.ds(0, FUTW)])
    for k in range(FUTW // 16):
        ids = _ids_from(sv[pl.ds(k * 16, 16)], tv[pl.ds(k * 16, 16)])
        idx2[0, pl.ds(k * 16, 16)] = ids
    pltpu.async_copy(emb.at[idx2.at[0]], rows.at[pl.ds(0, FUTW)], dsem).wait()
    pltpu.sync_copy(rows.at[pl.ds(0, FUTW)], outf.at[pl.ds(fbase, FUTW)])


def kernel(emb, sem_ids, token_type_ids, seq_mask, sem_ids_fut, token_type_ids_fut):
    sem = sem_ids.reshape(-1)
    tt = token_type_ids.reshape(-1)
    msk = seq_mask.astype(jnp.int32).reshape(-1)
    semf = sem_ids_fut.reshape(-1)
    ttf = token_type_ids_fut.reshape(-1)
    out, outf = _sc_embed(emb, sem, tt, msk, semf, ttf)
    return out.reshape(B, L, EMB_DIM), outf.reshape(B, LF, EMB_DIM)


# speedup vs baseline: 1.0540x; 1.0540x over previous
import functools

import jax
import jax.numpy as jnp
from jax import lax
from jax.experimental import pallas as pl
from jax.experimental.pallas import tpu as pltpu
from jax.experimental.pallas import tpu_sc as plsc

NUM_EMB = 100000
SEM_DIM = 4
EMB_DIM = 64
PAD = NUM_EMB * SEM_DIM
MAXV = PAD - 1
B = 1024
L = 800
LF = 4

NC = 2
NS = 16
W = NC * NS
MAIN = B * L
PW = MAIN // W
C = 640
G = 128
KC = C // G
NCH = PW // C
FUTW = (B * LF) // W


def _ids_from(s, t, m=None):
    t = jnp.minimum(jnp.maximum(t, jnp.int32(0)), jnp.int32(SEM_DIM - 1))
    ids = t * jnp.int32(NUM_EMB) + s
    inv = ((ids > jnp.int32(MAXV)) | (ids < jnp.int32(0))) & (s != jnp.int32(-1))
    ids = jnp.where(inv, jnp.int32(PAD), ids)
    if m is not None:
        ids = jnp.where(m != jnp.int32(0), ids, jnp.int32(PAD))
    return ids


@functools.partial(
    pl.kernel,
    mesh=plsc.VectorSubcoreMesh(core_axis_name="c", subcore_axis_name="s"),
    compiler_params=pltpu.CompilerParams(use_tc_tiling_on_sc=False),
    out_type=(
        jax.ShapeDtypeStruct((MAIN, EMB_DIM), jnp.float32),
        jax.ShapeDtypeStruct((B * LF, EMB_DIM), jnp.float32),
    ),
    scratch_types=[
        pltpu.VMEM((2, C), jnp.int32),
        pltpu.VMEM((2, C), jnp.int32),
        pltpu.VMEM((2, C), jnp.int32),
        pltpu.VMEM((2, KC, G), jnp.int32),
        pltpu.VMEM((2, C, EMB_DIM), jnp.float32),
        pltpu.SemaphoreType.DMA,
        pltpu.SemaphoreType.DMA,
        pltpu.SemaphoreType.DMA,
        pltpu.SemaphoreType.DMA,
    ],
)
def _sc_embed(emb, sem, tt, msk, semf, ttf, out, outf,
              sv, tv, mv, idx2, rows, lsem, gsem, s0sem, s1sem):
    wid = lax.axis_index("s") * NC + lax.axis_index("c")
    base = wid * PW
    ssem = (s0sem, s1sem)

    def fire_load(i, p):
        off = base + i * C
        pltpu.async_copy(sem.at[pl.ds(off, C)], sv.at[p], lsem)
        pltpu.async_copy(tt.at[pl.ds(off, C)], tv.at[p], lsem)
        pltpu.async_copy(msk.at[pl.ds(off, C)], mv.at[p], lsem)

    def wait_load(i, p):
        off = base + i * C
        pltpu.make_async_copy(sem.at[pl.ds(off, C)], sv.at[p], lsem).wait()
        pltpu.make_async_copy(tt.at[pl.ds(off, C)], tv.at[p], lsem).wait()
        pltpu.make_async_copy(msk.at[pl.ds(off, C)], mv.at[p], lsem).wait()

    def compute_idx(p):
        for j in range(KC):
            for k in range(G // 16):
                o = j * G + k * 16
                ids = _ids_from(sv[p, pl.ds(o, 16)], tv[p, pl.ds(o, 16)],
                                mv[p, pl.ds(o, 16)])
                idx2[p, j, pl.ds(k * 16, 16)] = ids

    def fire_gather(p):
        for j in range(KC):
            pltpu.async_copy(emb.at[idx2.at[p].at[j]],
                             rows.at[p].at[pl.ds(j * G, G)], gsem)

    def drain_gather(p):
        for j in range(KC):
            pltpu.make_async_copy(emb.at[idx2.at[p].at[j]],
                                  rows.at[p].at[pl.ds(j * G, G)], gsem).wait()

    def fire_store(i, p):
        off = base + i * C
        pltpu.async_copy(rows.at[p], out.at[pl.ds(off, C)], ssem[p])

    def wait_store(i, p):
        off = base + i * C
        pltpu.make_async_copy(rows.at[p], out.at[pl.ds(off, C)], ssem[p]).wait()

    fire_load(0, 0)

    def step(t, carry):
        for p in (0, 1):
            i = 2 * t + p
            wait_load(i, p)
            compute_idx(p)
            if p == 0:
                @pl.when(t > 0)
                def _():
                    wait_store(i - 2, 0)
                    drain_gather(1)
                    fire_store(i - 1, 1)
            else:
                @pl.when(t > 0)
                def _():
                    wait_store(i - 2, 1)
                drain_gather(0)
                fire_store(i - 1, 0)
            fire_gather(p)
            if p == 1:
                @pl.when(t < NCH // 2 - 1)
                def _():
                    fire_load(i + 1, 0)
            else:
                fire_load(i + 1, 1)
        return carry

    lax.fori_loop(0, NCH // 2, step, 0)

    wait_store(NCH - 2, 0)
    drain_gather(1)
    fire_store(NCH - 1, 1)

    fbase = wid * FUTW
    pltpu.sync_copy(semf.at[pl.ds(fbase, FUTW)], sv.at[0].at[pl.ds(0, FUTW)])
    pltpu.sync_copy(ttf.at[pl.ds(fbase, FUTW)], tv.at[0].at[pl.ds(0, FUTW)])
    for k in range(FUTW // 16):
        ids = _ids_from(sv[0, pl.ds(k * 16, 16)], tv[0, pl.ds(k * 16, 16)])
        idx2[0, 0, pl.ds(k * 16, 16)] = ids
    pltpu.async_copy(emb.at[idx2.at[0].at[0]],
                     rows.at[0].at[pl.ds(0, FUTW)], gsem).wait()
    pltpu.sync_copy(rows.at[0].at[pl.ds(0, FUTW)], outf.at[pl.ds(fbase, FUTW)])
    wait_store(NCH - 1, 1)


def kernel(emb, sem_ids, token_type_ids, seq_mask, sem_ids_fut, token_type_ids_fut):
    sem = sem_ids.reshape(-1)
    tt = token_type_ids.reshape(-1)
    msk = seq_mask.astype(jnp.int32).reshape(-1)
    semf = sem_ids_fut.reshape(-1)
    ttf = token_type_ids_fut.reshape(-1)
    out, outf = _sc_embed(emb, sem, tt, msk, semf, ttf)
    return out.reshape(B, L, EMB_DIM), outf.reshape(B, LF, EMB_DIM)
